# Initial kernel scaffold; baseline (speedup 1.0000x reference)
import jax
import jax.numpy as jnp
from jax.experimental import pallas as pl


def kernel(x, Wr, br, W1, b1, W2, b2):
    raise NotImplementedError("write your pallas kernel here")


# speedup vs baseline: 8.8943x; 8.8943x over previous
import functools

import jax
import jax.numpy as jnp
from jax import lax
from jax.experimental import pallas as pl
from jax.experimental.pallas import tpu as pltpu
from jax.experimental.pallas import tpu_sc as plsc

E = 64
K = 2
CAP = 512
DIN = 1024
DHID = 1024
DOUT = 1024
NTOK = 8192
TBLK = 1024
NB = NTOK // TBLK
SLOTS = E * CAP
XG_ROWS = SLOTS + CAP

NW = 32
TOK_PER_W = NTOK // NW
CHUNK = 32
NCHUNK = TOK_PER_W // CHUNK


def _router_body(x_ref, wr_ref, br_ref,
                 slot_s_ref, slot_g_ref, gate_ref, counts_ref, aux_ref,
                 psum_ref):
    b = pl.program_id(0)

    @pl.when(b == 0)
    def _():
        counts_ref[...] = jnp.zeros_like(counts_ref)
        psum_ref[...] = jnp.zeros_like(psum_ref)

    x = x_ref[...]
    logits = jnp.dot(x, wr_ref[...], preferred_element_type=jnp.float32)
    logits = logits + br_ref[...]
    m = jnp.max(logits, axis=1, keepdims=True)
    ex = jnp.exp(logits - m)
    probs = ex / jnp.sum(ex, axis=1, keepdims=True)
    psum_ref[...] += jnp.sum(probs, axis=0, keepdims=True)

    lane = lax.broadcasted_iota(jnp.int32, (TBLK, E), 1)
    m1 = jnp.max(probs, axis=1, keepdims=True)
    idx1 = jnp.min(jnp.where(probs == m1, lane, E), axis=1, keepdims=True)
    pm = jnp.where(lane == idx1, -1.0, probs)
    m2 = jnp.max(pm, axis=1, keepdims=True)
    idx2 = jnp.min(jnp.where(pm == m2, lane, E), axis=1, keepdims=True)

    oh = ((lane == idx1) | (lane == idx2)).astype(jnp.float32)
    base = counts_ref[...]
    r = lax.broadcasted_iota(jnp.int32, (TBLK, TBLK), 0)
    c = lax.broadcasted_iota(jnp.int32, (TBLK, TBLK), 1)
    tri = (c < r).astype(jnp.float32)
    csum_ex = jnp.dot(tri, oh, preferred_element_type=jnp.float32)
    posf = csum_ex + base
    counts_ref[...] = base + jnp.sum(oh, axis=0, keepdims=True)

    for k, (idxk, mk) in enumerate(((idx1, m1), (idx2, m2))):
        pk = jnp.sum(jnp.where(lane == idxk, posf, 0.0),
                     axis=1, keepdims=True).astype(jnp.int32)
        slotv = idxk * CAP + pk
        valid = pk < CAP
        ss = jnp.where(valid, slotv, SLOTS)
        sg = jnp.where(valid, slotv, 0)
        gg = jnp.where(valid, mk, 0.0)
        slot_s_ref[0, k, :] = jnp.transpose(ss)[0]
        slot_g_ref[0, k, :] = jnp.transpose(sg)[0]
        gate_ref[0, k, :, :] = jnp.broadcast_to(gg, (TBLK, 16))

    @pl.when(b == NB - 1)
    def _():
        cnts = counts_ref[...]
        ps = psum_ref[...]
        balance = jnp.sum(ps / NTOK * (cnts / NTOK)) * E
        imp = jnp.sum(ps * ps) / E
        aux_ref[...] = jnp.reshape(balance + imp, (1, 1))


def _run_router(x, Wr, br):
    br2 = br.reshape(1, E)
    out_shape = [
        jax.ShapeDtypeStruct((NB, K, TBLK), jnp.int32),
        jax.ShapeDtypeStruct((NB, K, TBLK), jnp.int32),
        jax.ShapeDtypeStruct((NB, K, TBLK, 16), jnp.float32),
        jax.ShapeDtypeStruct((1, E), jnp.float32),
        jax.ShapeDtypeStruct((1, 1), jnp.float32),
    ]
    return pl.pallas_call(
        _router_body,
        grid=(NB,),
        in_specs=[
            pl.BlockSpec((TBLK, DIN), lambda b: (b, 0)),
            pl.BlockSpec((DIN, E), lambda b: (0, 0)),
            pl.BlockSpec((1, E), lambda b: (0, 0)),
        ],
        out_specs=[
            pl.BlockSpec((1, K, TBLK), lambda b: (b, 0, 0)),
            pl.BlockSpec((1, K, TBLK), lambda b: (b, 0, 0)),
            pl.BlockSpec((1, K, TBLK, 16), lambda b: (b, 0, 0, 0)),
            pl.BlockSpec((1, E), lambda b: (0, 0)),
            pl.BlockSpec((1, 1), lambda b: (0, 0)),
        ],
        out_shape=out_shape,
        scratch_shapes=[pltpu.VMEM((1, E), jnp.float32)],
    )(x, Wr, br2)


def _dispatch(x, slot_s2):
    mesh = plsc.VectorSubcoreMesh(core_axis_name="c", subcore_axis_name="s")

    @functools.partial(
        pl.kernel,
        mesh=mesh,
        out_type=jax.ShapeDtypeStruct((XG_ROWS, DIN), jnp.float32),
        scratch_types=[
            pltpu.VMEM((CHUNK, DIN), jnp.float32),
            pltpu.VMEM((CHUNK,), jnp.int32),
            pltpu.VMEM((CHUNK,), jnp.int32),
            pltpu.SemaphoreType.DMA,
        ],
    )
    def body(x_hbm, slot_hbm, xg_hbm, xbuf, idx0, idx1, sem):
        wid = lax.axis_index("s") * 2 + lax.axis_index("c")
        for ci in range(NCHUNK):
            t0 = wid * TOK_PER_W + ci * CHUNK
            pltpu.sync_copy(x_hbm.at[pl.ds(t0, CHUNK)], xbuf)
            pltpu.sync_copy(slot_hbm.at[0, pl.ds(t0, CHUNK)], idx0)
            pltpu.sync_copy(slot_hbm.at[1, pl.ds(t0, CHUNK)], idx1)
            pltpu.async_copy(xbuf, xg_hbm.at[idx0], sem).wait()
            pltpu.async_copy(xbuf, xg_hbm.at[idx1], sem).wait()

    return body(x, slot_s2)


def _erf(z):
    return lax.erf(z)


def _ffn_body(cnt_ref, xg_ref, w1_ref, b1_ref, w2_ref, b2_ref, yg_ref):
    e = pl.program_id(0)
    cnt = cnt_ref[e]
    rowmask = lax.broadcasted_iota(jnp.int32, (CAP, 1), 0) < cnt
    xb = jnp.where(rowmask, xg_ref[...], 0.0)
    h = jnp.dot(xb, w1_ref[0], preferred_element_type=jnp.float32) + b1_ref[0]
    h = 0.5 * h * (1.0 + _erf(h * 0.7071067811865476))
    y = jnp.dot(h, w2_ref[0], preferred_element_type=jnp.float32) + b2_ref[0]
    yg_ref[...] = y


def _run_ffn(counts_i, xg, W1, b1, W2, b2):
    grid_spec = pltpu.PrefetchScalarGridSpec(
        num_scalar_prefetch=1,
        grid=(E,),
        in_specs=[
            pl.BlockSpec((CAP, DIN), lambda e, c: (e, 0)),
            pl.BlockSpec((1, DIN, DHID), lambda e, c: (e, 0, 0)),
            pl.BlockSpec((1, 1, DHID), lambda e, c: (e, 0, 0)),
            pl.BlockSpec((1, DHID, DOUT), lambda e, c: (e, 0, 0)),
            pl.BlockSpec((1, 1, DOUT), lambda e, c: (e, 0, 0)),
        ],
        out_specs=pl.BlockSpec((CAP, DOUT), lambda e, c: (e, 0)),
    )
    return pl.pallas_call(
        _ffn_body,
        grid_spec=grid_spec,
        out_shape=jax.ShapeDtypeStruct((SLOTS, DOUT), jnp.float32),
    )(counts_i, xg, W1, b1.reshape(E, 1, DHID), W2, b2.reshape(E, 1, DOUT))


def _combine(yg, slot_g2, gates2):
    mesh = plsc.VectorSubcoreMesh(core_axis_name="c", subcore_axis_name="s")
    nvec = DOUT // 16

    @functools.partial(
        pl.kernel,
        mesh=mesh,
        out_type=jax.ShapeDtypeStruct((NTOK, DOUT), jnp.float32),
        scratch_types=[
            pltpu.VMEM((CHUNK, DOUT), jnp.float32),
            pltpu.VMEM((CHUNK, DOUT), jnp.float32),
            pltpu.VMEM((CHUNK,), jnp.int32),
            pltpu.VMEM((CHUNK,), jnp.int32),
            pltpu.VMEM((CHUNK, 16), jnp.float32),
            pltpu.VMEM((CHUNK, 16), jnp.float32),
            pltpu.SemaphoreType.DMA,
        ],
    )
    def body(yg_hbm, slot_hbm, gate_hbm, out_hbm,
             buf0, buf1, idx0, idx1, g0b, g1b, sem):
        wid = lax.axis_index("s") * 2 + lax.axis_index("c")
        for ci in range(NCHUNK):
            t0 = wid * TOK_PER_W + ci * CHUNK
            pltpu.sync_copy(slot_hbm.at[0, pl.ds(t0, CHUNK)], idx0)
            pltpu.sync_copy(slot_hbm.at[1, pl.ds(t0, CHUNK)], idx1)
            pltpu.sync_copy(gate_hbm.at[0, pl.ds(t0, CHUNK)], g0b)
            pltpu.sync_copy(gate_hbm.at[1, pl.ds(t0, CHUNK)], g1b)
            pltpu.async_copy(yg_hbm.at[idx0], buf0, sem).wait()
            pltpu.async_copy(yg_hbm.at[idx1], buf1, sem).wait()

            def row_body(j, _):
                g0 = g0b[j, :]
                g1 = g1b[j, :]

                def v_body(v, _):
                    s = pl.ds(v * 16, 16)
                    buf0[j, s] = g0 * buf0[j, s] + g1 * buf1[j, s]
                    return 0

                lax.fori_loop(0, nvec, v_body, 0)
                return 0

            lax.fori_loop(0, CHUNK, row_body, 0)
            pltpu.sync_copy(buf0, out_hbm.at[pl.ds(t0, CHUNK)])

    return body(yg, slot_g2, gates2)


def kernel(x, Wr, br, W1, b1, W2, b2):
    slot_s, slot_g, gates, counts, aux = _run_router(x, Wr, br)
    slot_s2 = slot_s.transpose(1, 0, 2).reshape(K, NTOK)
    slot_g2 = slot_g.transpose(1, 0, 2).reshape(K, NTOK)
    gates2 = gates.transpose(1, 0, 2, 3).reshape(K, NTOK, 16)
    counts_i = counts[0].astype(jnp.int32)
    xg = _dispatch(x, slot_s2)
    yg = _run_ffn(counts_i, xg, W1, b1, W2, b2)
    final = _combine(yg, slot_g2, gates2)
    return final, aux[0, 0]
